# baseline (speedup 1.0000x reference)
import functools

import jax
import jax.numpy as jnp
from jax.experimental import pallas as pl
from jax.experimental.pallas import tpu as pltpu

T_BLOCK = 512
E = 64
TOP_K = 8
N_GROUPS = 8
GROUP = E // N_GROUPS
ROUTED_SCALING_FACTOR = 2.5


def _partner(x, k):
    lane = jax.lax.broadcasted_iota(jnp.int32, x.shape, x.ndim - 1)
    bit = (lane & k) != 0
    n = x.shape[-1]
    return jnp.where(bit, pltpu.roll(x, k, axis=x.ndim - 1),
                     pltpu.roll(x, n - k, axis=x.ndim - 1))


def _router_body(x_ref, w_ref, b_ref, wout_ref, iout_ref):
    x = x_ref[...]
    w = w_ref[...]
    scores = jax.lax.dot_general(
        x, w, (((1,), (0,)), ((), ())),
        preferred_element_type=jnp.float32,
        precision=jax.lax.Precision.DEFAULT)
    scores = 1.0 / (1.0 + jnp.exp(-scores))
    sb = scores + b_ref[...]

    lane = jax.lax.broadcasted_iota(jnp.int32, sb.shape, 1)

    m1 = sb
    m2 = jnp.full_like(sb, -jnp.inf)
    for k in (1, 2, 4):
        p1 = _partner(m1, k)
        p2 = _partner(m2, k)
        new_m1 = jnp.maximum(m1, p1)
        new_m2 = jnp.maximum(jnp.minimum(m1, p1), jnp.maximum(m2, p2))
        m1, m2 = new_m1, new_m2
    gs = m1 + m2

    beat_cnt = jnp.zeros_like(lane)
    for d in range(1, N_GROUPS):
        other = pltpu.roll(gs, GROUP * d, axis=1)
        tie_lower = (lane // GROUP) >= d
        beats = (other > gs) | ((other == gs) & tie_lower)
        beat_cnt = beat_cnt + beats.astype(jnp.int32)
    group_sel = beat_cnt < 4
    masked = jnp.where(group_sel, sb, 0.0)

    work = masked
    idx_cols = []
    w_cols = []
    for _ in range(TOP_K):
        m = jnp.max(work, axis=1, keepdims=True)
        eq = work == m
        idx = jnp.min(jnp.where(eq, lane, E), axis=1, keepdims=True)
        onehot = lane == idx
        w_cols.append(jnp.sum(jnp.where(onehot, scores, 0.0), axis=1,
                              keepdims=True))
        idx_cols.append(idx)
        work = jnp.where(onehot, -jnp.inf, work)

    weights = jnp.concatenate(w_cols, axis=1)
    weights = weights / (jnp.sum(weights, axis=1, keepdims=True) + 1e-20)
    weights = weights * ROUTED_SCALING_FACTOR
    wout_ref[...] = weights
    iout_ref[...] = jnp.concatenate(idx_cols, axis=1).astype(jnp.int32)


@jax.jit
def kernel(x_TD, kernel_DE, bias_E):
    T, D = x_TD.shape
    bt = T_BLOCK if T % T_BLOCK == 0 else T
    wout, iout = pl.pallas_call(
        _router_body,
        grid=(T // bt,),
        in_specs=[
            pl.BlockSpec((bt, D), lambda i: (i, 0)),
            pl.BlockSpec((D, E), lambda i: (0, 0)),
            pl.BlockSpec((1, E), lambda i: (0, 0)),
        ],
        out_specs=[
            pl.BlockSpec((bt, TOP_K), lambda i: (i, 0)),
            pl.BlockSpec((bt, TOP_K), lambda i: (i, 0)),
        ],
        out_shape=[
            jax.ShapeDtypeStruct((T, TOP_K), jnp.float32),
            jax.ShapeDtypeStruct((T, TOP_K), jnp.int32),
        ],
        compiler_params=pltpu.CompilerParams(
            dimension_semantics=("arbitrary",)),
    )(x_TD.astype(jnp.float32), kernel_DE, bias_E.reshape(1, E))
    return (wout, iout)


# speedup vs baseline: 2.6040x; 2.6040x over previous
import jax
import jax.numpy as jnp
from jax.experimental import pallas as pl
from jax.experimental.pallas import tpu as pltpu

T_BLOCK = 512
E = 64
TOP_K = 8
N_GROUPS = 8
GROUP = E // N_GROUPS
ROUTED_SCALING_FACTOR = 2.5


def _partner(x, k):
    row = jax.lax.broadcasted_iota(jnp.int32, x.shape, 0)
    bit = (row & k) != 0
    n = x.shape[0]
    return jnp.where(bit, pltpu.roll(x, k, axis=0),
                     pltpu.roll(x, n - k, axis=0))


def _route_tile(st, bias_col):
    row = jax.lax.broadcasted_iota(jnp.int32, st.shape, 0)
    scores = 1.0 / (1.0 + jnp.exp(-st))
    sb = scores + bias_col

    m1 = sb
    m2 = jnp.full_like(sb, -jnp.inf)
    for k in (1, 2, 4):
        p1 = _partner(m1, k)
        p2 = _partner(m2, k)
        m1, m2 = (jnp.maximum(m1, p1),
                  jnp.maximum(jnp.minimum(m1, p1), jnp.maximum(m2, p2)))
    gs = m1 + m2

    beat_cnt = jnp.zeros_like(row)
    for d in range(1, N_GROUPS):
        other = pltpu.roll(gs, GROUP * d, axis=0)
        tie_lower = (row // GROUP) >= d
        beats = (other > gs) | ((other == gs) & tie_lower)
        beat_cnt = beat_cnt + beats.astype(jnp.int32)
    masked = jnp.where(beat_cnt < 4, sb, 0.0)

    work = masked
    idx_rows = []
    w_rows = []
    for _ in range(TOP_K):
        m = jnp.max(work, axis=0, keepdims=True)
        idx = jnp.min(jnp.where(work == m, row, E), axis=0, keepdims=True)
        onehot = row == idx
        w_rows.append(jnp.sum(jnp.where(onehot, scores, 0.0), axis=0,
                              keepdims=True))
        idx_rows.append(idx)
        work = jnp.where(onehot, -jnp.inf, work)

    weights = jnp.concatenate(w_rows, axis=0)
    weights = weights / (jnp.sum(weights, axis=0, keepdims=True) + 1e-20)
    weights = weights * ROUTED_SCALING_FACTOR
    indices = jnp.concatenate(idx_rows, axis=0).astype(jnp.int32)
    return weights, indices


def _router_body(x_ref, w_ref, b_ref, wout_ref, iout_ref, scratch_ref):
    i = pl.program_id(0)
    n = pl.num_programs(0)

    @pl.when(i < n - 1)
    def _produce():
        s = jax.lax.dot_general(
            x_ref[...], w_ref[...], (((1,), (0,)), ((), ())),
            preferred_element_type=jnp.float32,
            precision=jax.lax.Precision.DEFAULT)
        scratch_ref[i % 2] = s.T

    @pl.when(i > 0)
    def _consume():
        weights, indices = _route_tile(scratch_ref[(i - 1) % 2], b_ref[...])
        wout_ref[...] = weights
        iout_ref[...] = indices


@jax.jit
def kernel(x_TD, kernel_DE, bias_E):
    T, D = x_TD.shape
    bt = T_BLOCK if T % T_BLOCK == 0 else T
    n_blocks = T // bt
    w8T, i8T = pl.pallas_call(
        _router_body,
        grid=(n_blocks + 1,),
        in_specs=[
            pl.BlockSpec((bt, D), lambda i: (jnp.minimum(i, pl.num_programs(0) - 2), 0)),
            pl.BlockSpec((D, E), lambda i: (0, 0)),
            pl.BlockSpec((E, 1), lambda i: (0, 0)),
        ],
        out_specs=[
            pl.BlockSpec((TOP_K, bt), lambda i: (0, jnp.maximum(i - 1, 0))),
            pl.BlockSpec((TOP_K, bt), lambda i: (0, jnp.maximum(i - 1, 0))),
        ],
        out_shape=[
            jax.ShapeDtypeStruct((TOP_K, T), jnp.float32),
            jax.ShapeDtypeStruct((TOP_K, T), jnp.int32),
        ],
        scratch_shapes=[pltpu.VMEM((2, E, bt), jnp.float32)],
        compiler_params=pltpu.CompilerParams(
            dimension_semantics=("arbitrary",)),
    )(x_TD.astype(jnp.float32), kernel_DE, bias_E.reshape(E, 1))
    return (w8T.T, i8T.T)


# speedup vs baseline: 2.8629x; 1.0994x over previous
import jax
import jax.numpy as jnp
from jax.experimental import pallas as pl
from jax.experimental.pallas import tpu as pltpu

T_BLOCK = 1024
E = 64
TOP_K = 8
N_GROUPS = 8
GROUP = E // N_GROUPS
ROUTED_SCALING_FACTOR = 2.5


def _partner(x, k):
    row = jax.lax.broadcasted_iota(jnp.int32, x.shape, 0)
    bit = (row & k) != 0
    n = x.shape[0]
    return jnp.where(bit, pltpu.roll(x, k, axis=0),
                     pltpu.roll(x, n - k, axis=0))


def _route_tile(st, bias_col):
    row = jax.lax.broadcasted_iota(jnp.int32, st.shape, 0)
    scores = 1.0 / (1.0 + jnp.exp(-st))
    sb = scores + bias_col

    m1 = sb
    m2 = jnp.full_like(sb, -jnp.inf)
    for k in (1, 2, 4):
        p1 = _partner(m1, k)
        p2 = _partner(m2, k)
        m1, m2 = (jnp.maximum(m1, p1),
                  jnp.maximum(jnp.minimum(m1, p1), jnp.maximum(m2, p2)))
    gs = m1 + m2

    beat_cnt = jnp.zeros_like(row)
    for d in range(1, N_GROUPS):
        other = pltpu.roll(gs, GROUP * d, axis=0)
        tie_lower = (row // GROUP) >= d
        beats = (other > gs) | ((other == gs) & tie_lower)
        beat_cnt = beat_cnt + beats.astype(jnp.int32)
    masked = jnp.where(beat_cnt < 4, sb, 0.0)

    work = masked
    idx_rows = []
    w_rows = []
    for _ in range(TOP_K):
        m = jnp.max(work, axis=0, keepdims=True)
        idx = jnp.min(jnp.where(work == m, row, E), axis=0, keepdims=True)
        onehot = row == idx
        w_rows.append(jnp.sum(jnp.where(onehot, scores, 0.0), axis=0,
                              keepdims=True))
        idx_rows.append(idx)
        work = jnp.where(onehot, -jnp.inf, work)

    weights = jnp.concatenate(w_rows, axis=0)
    weights = weights / (jnp.sum(weights, axis=0, keepdims=True) + 1e-20)
    weights = weights * ROUTED_SCALING_FACTOR
    indices = jnp.concatenate(idx_rows, axis=0).astype(jnp.int32)
    return weights, indices


def _router_body(x_ref, w_ref, b_ref, wout_ref, iout_ref, scratch_ref):
    i = pl.program_id(0)
    n = pl.num_programs(0)

    @pl.when(i < n - 1)
    def _produce():
        s = jax.lax.dot_general(
            x_ref[...], w_ref[...], (((1,), (0,)), ((), ())),
            preferred_element_type=jnp.float32,
            precision=jax.lax.Precision.DEFAULT)
        scratch_ref[i % 2] = s.T

    @pl.when(i > 0)
    def _consume():
        weights, indices = _route_tile(scratch_ref[(i - 1) % 2], b_ref[...])
        wout_ref[...] = weights
        iout_ref[...] = indices


@jax.jit
def kernel(x_TD, kernel_DE, bias_E):
    T, D = x_TD.shape
    bt = T_BLOCK if T % T_BLOCK == 0 else T
    n_blocks = T // bt
    w8T, i8T = pl.pallas_call(
        _router_body,
        grid=(n_blocks + 1,),
        in_specs=[
            pl.BlockSpec((bt, D), lambda i: (jnp.minimum(i, pl.num_programs(0) - 2), 0)),
            pl.BlockSpec((D, E), lambda i: (0, 0)),
            pl.BlockSpec((E, 1), lambda i: (0, 0)),
        ],
        out_specs=[
            pl.BlockSpec((TOP_K, bt), lambda i: (0, jnp.maximum(i - 1, 0))),
            pl.BlockSpec((TOP_K, bt), lambda i: (0, jnp.maximum(i - 1, 0))),
        ],
        out_shape=[
            jax.ShapeDtypeStruct((TOP_K, T), jnp.float32),
            jax.ShapeDtypeStruct((TOP_K, T), jnp.int32),
        ],
        scratch_shapes=[pltpu.VMEM((2, E, bt), jnp.float32)],
        compiler_params=pltpu.CompilerParams(
            dimension_semantics=("arbitrary",)),
    )(x_TD.astype(jnp.float32), kernel_DE, bias_E.reshape(E, 1))
    return (w8T.T, i8T.T)
